# baseline (speedup 1.0000x reference)
import functools

import jax
import jax.numpy as jnp
from jax import lax
from jax.experimental import pallas as pl
from jax.experimental.pallas import tpu as pltpu
from jax.experimental.pallas import tpu_sc as plsc

VOCAB = 1000000
EMBED_DIM = 64
BATCH = 4096
HIST_LEN = 200
DROPOUT = 0.1

NUM_IDX = BATCH * HIST_LEN
NC = 2
NS = 16
NW = NC * NS
PER_W = NUM_IDX // NW
CHUNK = 128
CHUNKS = PER_W // CHUNK
LANES = 16


def _emb_dropout_call(weight, scale, idx):
    mesh = plsc.VectorSubcoreMesh(core_axis_name="c", subcore_axis_name="s")

    @functools.partial(
        pl.kernel,
        out_type=jax.ShapeDtypeStruct((NUM_IDX, EMBED_DIM), jnp.float32),
        mesh=mesh,
        scratch_types=[
            pltpu.VMEM((CHUNKS, CHUNK), jnp.int32),
            pltpu.VMEM((CHUNK,), jnp.float32),
            pltpu.VMEM((CHUNK, EMBED_DIM), jnp.float32),
            pltpu.SemaphoreType.DMA,
        ],
        compiler_params=pltpu.CompilerParams(
            needs_layout_passes=False, use_tc_tiling_on_sc=False
        ),
    )
    def kern(weight_hbm, scale_hbm, idx_hbm, out_hbm, idx_v, scl_v, rows_v, sem):
        wid = lax.axis_index("s") * NC + lax.axis_index("c")
        pltpu.sync_copy(idx_hbm.at[wid], idx_v)
        out_base = wid * PER_W

        def chunk_body(j, carry):
            cr = pltpu.async_copy(weight_hbm.at[idx_v.at[j]], rows_v, sem)
            cs = pltpu.async_copy(scale_hbm.at[idx_v.at[j]], scl_v, sem)
            cr.wait()
            cs.wait()

            def mul16(g, c2):
                base_r = g * LANES
                for t in range(LANES):
                    r = base_r + t
                    bs = plsc.load_gather(
                        scl_v, [jnp.full((LANES,), r, dtype=jnp.int32)]
                    )
                    for kk in range(EMBED_DIM // LANES):
                        sl = pl.ds(kk * LANES, LANES)
                        rows_v[r, sl] = rows_v[r, sl] * bs
                return c2

            lax.fori_loop(0, CHUNK // LANES, mul16, 0)
            pltpu.sync_copy(rows_v, out_hbm.at[pl.ds(out_base + j * CHUNK, CHUNK)])
            return carry

        lax.fori_loop(0, CHUNKS, chunk_body, 0)

    return kern(weight, scale, idx)


def kernel(weight, words):
    keep = jax.random.bernoulli(
        jax.random.key(42), 1.0 - DROPOUT, (weight.shape[0], 1)
    )
    scale = keep.astype(weight.dtype).reshape(VOCAB) / (1.0 - DROPOUT)
    idx = words.astype(jnp.int32).reshape(NW, CHUNKS, CHUNK)
    out = _emb_dropout_call(weight, scale, idx)
    return out.reshape(BATCH, HIST_LEN, EMBED_DIM)


# speedup vs baseline: 1.4782x; 1.4782x over previous
import functools

import jax
import jax.numpy as jnp
from jax import lax
from jax.experimental import pallas as pl
from jax.experimental.pallas import tpu as pltpu
from jax.experimental.pallas import tpu_sc as plsc

VOCAB = 1000000
EMBED_DIM = 64
BATCH = 4096
HIST_LEN = 200
DROPOUT = 0.1

NUM_IDX = BATCH * HIST_LEN
NC = 2
NS = 16
NW = NC * NS
PER_W = NUM_IDX // NW
GATHER_W = 128
CHUNK = 256
SUB = CHUNK // GATHER_W
NCHUNK = PER_W // CHUNK
IDX_ROWS = PER_W // GATHER_W
LANES = 16

_BCAST_DNUMS = lax.GatherDimensionNumbers(
    offset_dims=(), collapsed_slice_dims=(0,), start_index_map=(0,)
)


def _bcast(vec, lane):
    idx = jnp.full((LANES, 1), lane, dtype=jnp.int32)
    return lax.gather(
        vec, idx, _BCAST_DNUMS, (1,),
        mode=lax.GatherScatterMode.PROMISE_IN_BOUNDS,
    )


def _emb_dropout_call(weight, scale, idx):
    mesh = plsc.VectorSubcoreMesh(core_axis_name="c", subcore_axis_name="s")

    @functools.partial(
        pl.kernel,
        out_type=jax.ShapeDtypeStruct((NUM_IDX, EMBED_DIM), jnp.float32),
        mesh=mesh,
        scratch_types=[
            pltpu.VMEM((IDX_ROWS, GATHER_W), jnp.int32),
            pltpu.VMEM((2, CHUNK), jnp.float32),
            pltpu.VMEM((2, CHUNK, EMBED_DIM), jnp.float32),
            pltpu.VMEM((2, CHUNK, EMBED_DIM), jnp.float32),
            pltpu.SemaphoreType.DMA,
            pltpu.SemaphoreType.DMA,
            pltpu.SemaphoreType.DMA,
            pltpu.SemaphoreType.DMA,
        ],
        compiler_params=pltpu.CompilerParams(
            needs_layout_passes=False, use_tc_tiling_on_sc=False
        ),
    )
    def kern(weight_hbm, scale_hbm, idx_hbm, out_hbm,
             idx_v, scl_v, inb_v, outb_v, g0, g1, o0, o1):
        wid = lax.axis_index("s") * NC + lax.axis_index("c")
        pltpu.sync_copy(idx_hbm.at[wid], idx_v)
        out_base = wid * PER_W
        gsem = (g0, g1)
        osem = (o0, o1)

        def fire_gather(c, b):
            for k in range(SUB):
                irow = idx_v.at[c * SUB + k]
                pltpu.async_copy(
                    weight_hbm.at[irow],
                    inb_v.at[b].at[pl.ds(k * GATHER_W, GATHER_W)],
                    gsem[b],
                )
                pltpu.async_copy(
                    scale_hbm.at[irow],
                    scl_v.at[b].at[pl.ds(k * GATHER_W, GATHER_W)],
                    gsem[b],
                )

        def wait_gather(b):
            pltpu.make_async_copy(
                weight_hbm.at[pl.ds(0, CHUNK)], inb_v.at[b], gsem[b]
            ).wait()
            pltpu.make_async_copy(
                scale_hbm.at[pl.ds(0, CHUNK)], scl_v.at[b], gsem[b]
            ).wait()

        def fire_out(c, b):
            pltpu.async_copy(
                outb_v.at[b],
                out_hbm.at[pl.ds(out_base + c * CHUNK, CHUNK)],
                osem[b],
            )

        def wait_out(b):
            pltpu.make_async_copy(
                outb_v.at[b], out_hbm.at[pl.ds(0, CHUNK)], osem[b]
            ).wait()

        def mul(b):
            src = inb_v.at[b]
            dst = outb_v.at[b]
            scl = scl_v.at[b]

            def mul16(g, carry):
                sv = scl[pl.ds(g * LANES, LANES)]
                base_r = g * LANES
                for t in range(LANES):
                    bs = _bcast(sv, t)
                    r = base_r + t
                    for kk in range(EMBED_DIM // LANES):
                        sl = pl.ds(kk * LANES, LANES)
                        dst[r, sl] = src[r, sl] * bs
                return carry

            lax.fori_loop(0, CHUNK // LANES, mul16, 0)

        fire_gather(0, 0)
        fire_gather(1, 1)

        def step(i2, carry):
            a = 2 * i2
            for b in range(2):
                c = a + b
                wait_gather(b)

                @pl.when(i2 > 0)
                def _():
                    wait_out(b)

                mul(b)
                fire_out(c, b)

                @pl.when(c + 2 < NCHUNK)
                def _():
                    fire_gather(c + 2, b)

            return carry

        lax.fori_loop(0, NCHUNK // 2, step, 0)
        wait_out(0)
        wait_out(1)

    return kern(weight, scale, idx)


def kernel(weight, words):
    keep = jax.random.bernoulli(
        jax.random.key(42), 1.0 - DROPOUT, (weight.shape[0], 1)
    )
    scale = keep.astype(weight.dtype).reshape(VOCAB) / (1.0 - DROPOUT)
    idx = words.astype(jnp.int32).reshape(NW, IDX_ROWS, GATHER_W)
    out = _emb_dropout_call(weight, scale, idx)
    return out.reshape(BATCH, HIST_LEN, EMBED_DIM)
